# baseline (speedup 1.0000x reference)
import functools
import jax
import jax.numpy as jnp
from jax import lax
from jax.experimental import pallas as pl
from jax.experimental.pallas import tpu as pltpu
from jax.experimental.pallas import tpu_sc as plsc

THRESHOLD = 0.12
RES = 64
NC, NS, L = 2, 16, 16
NW = NC * NS
J = 24
JP = 32
C = 128
G = C // L

_CORNERS = [(dx, dy, dz) for dx in (0, 1) for dy in (0, 1) for dz in (0, 1)]


def _splat_i32(x):
    return jnp.full((L,), x, jnp.int32)


_GDN = lax.GatherDimensionNumbers(offset_dims=(), collapsed_slice_dims=(0,),
                                  start_index_map=(0,))


def _vgather(vec, idx):
    return lax.gather(vec, idx[:, None], _GDN, (1,),
                      mode=lax.GatherScatterMode.PROMISE_IN_BOUNDS)


def _gather_rows(table_hbm, idx_ref, dst_ref, sem):
    return pltpu.make_async_copy(table_hbm.at[idx_ref], dst_ref, sem)


def _worker_id():
    return lax.axis_index("s") * NC + lax.axis_index("c")


def _sc_call(vox_t, pts3, bar3, idx3, so_flat, tfs2, cst, n, V):
    PT = n // NW
    NCHUNK = PT // C
    PB = PT // 128
    f32 = jnp.float32

    def body(vox_hbm, pts_hbm, bar_hbm, idx_hbm, so_hbm, tfs_hbm, cst_hbm,
             pdef_hbm, wtf_hbm,
             so_v, tfs_v, cst_v, ptsv, barv, idxv, idxbuf, gbuf, fbuf,
             pdef_v, wtf_v, sem):
        wid = _worker_id()
        base = wid * PT

        pltpu.sync_copy(so_hbm, so_v)
        pltpu.sync_copy(tfs_hbm, tfs_v)
        pltpu.sync_copy(cst_hbm, cst_v)
        for c in range(3):
            pltpu.sync_copy(pts_hbm.at[pl.ds(c * n + base, PT)],
                            ptsv.at[pl.ds(c * PT, PT)])
            pltpu.sync_copy(bar_hbm.at[pl.ds(c * n + base, PT)],
                            barv.at[pl.ds(c * PT, PT)])
            pltpu.sync_copy(idx_hbm.at[pl.ds(c * n + base, PT)],
                            idxv.at[pl.ds(c * PT, PT)])

        iota = lax.iota(jnp.int32, L)
        splats = [_splat_i32(a) for a in range(16)]
        cstv = cst_v[pl.ds(0, L)]
        sx = _vgather(cstv, splats[0])
        sy = _vgather(cstv, splats[1])
        sz = _vgather(cstv, splats[2])
        ox = _vgather(cstv, splats[3])
        oy = _vgather(cstv, splats[4])
        oz = _vgather(cstv, splats[5])

        def axis_prep(p, s, o):
            g = jnp.clip(p * s + o, 0.0, 1.0) * jnp.float32(RES - 1)
            gi = jnp.minimum(g.astype(jnp.int32), RES - 2)
            return gi, g - gi.astype(jnp.float32)

        def chunk(t, carry):
            cb = t * C

            def p1(i, c2):
                o = cb + i * L
                qx = ptsv[pl.ds(o, L)]
                qy = ptsv[pl.ds(PT + o, L)]
                qz = ptsv[pl.ds(2 * PT + o, L)]
                b0 = barv[pl.ds(o, L)]
                b1 = barv[pl.ds(PT + o, L)]
                b2 = barv[pl.ds(2 * PT + o, L)]
                i0 = idxv[pl.ds(o, L)]
                i1 = idxv[pl.ds(PT + o, L)]
                i2 = idxv[pl.ds(2 * PT + o, L)]
                px = qx + (b0 * plsc.load_gather(so_v, [i0])
                           + b1 * plsc.load_gather(so_v, [i1])
                           + b2 * plsc.load_gather(so_v, [i2]))
                py = qy + (b0 * plsc.load_gather(so_v, [i0 + V])
                           + b1 * plsc.load_gather(so_v, [i1 + V])
                           + b2 * plsc.load_gather(so_v, [i2 + V]))
                pz = qz + (b0 * plsc.load_gather(so_v, [i0 + 2 * V])
                           + b1 * plsc.load_gather(so_v, [i1 + 2 * V])
                           + b2 * plsc.load_gather(so_v, [i2 + 2 * V]))
                gix, fx = axis_prep(px, sx, ox)
                giy, fy = axis_prep(py, sy, oy)
                giz, fz = axis_prep(pz, sz, oz)
                rowb = (gix * RES + giy) * RES + giz
                oo = i * L
                for c, (dx, dy, dz) in enumerate(_CORNERS):
                    idxbuf[pl.ds(c * C + oo, L)] = rowb + (dx * RES * RES
                                                           + dy * RES + dz)
                fbuf[pl.ds(oo, L)] = fx
                fbuf[pl.ds(C + oo, L)] = fy
                fbuf[pl.ds(2 * C + oo, L)] = fz
                fbuf[pl.ds(3 * C + oo, L)] = px
                fbuf[pl.ds(4 * C + oo, L)] = py
                fbuf[pl.ds(5 * C + oo, L)] = pz
                return c2

            lax.fori_loop(0, G, p1, 0)

            cps = []
            for c in range(8):
                cps.append(_gather_rows(
                    vox_hbm, idxbuf.at[pl.ds(c * C, C)],
                    gbuf.at[pl.ds(c * C, C)], sem))
            for cp in cps:
                cp.start()
            for cp in cps:
                cp.wait()

            def p2(i, c2):
                oo = i * L
                fx = fbuf[pl.ds(oo, L)]
                fy = fbuf[pl.ds(C + oo, L)]
                fz = fbuf[pl.ds(2 * C + oo, L)]
                px = fbuf[pl.ds(3 * C + oo, L)]
                py = fbuf[pl.ds(4 * C + oo, L)]
                pz = fbuf[pl.ds(5 * C + oo, L)]
                ex = 1.0 - fx
                ey = 1.0 - fy
                ez = 1.0 - fz
                cxy = [ex * ey, ex * fy, fx * ey, fx * fy]
                coefs = []
                for c, (dx, dy, dz) in enumerate(_CORNERS):
                    coefs.append(cxy[dx * 2 + dy] * (fz if dz else ez))
                rows = [iota + (oo + c * C) for c in range(8)]
                s = jnp.full((L,), 1e-8, jnp.float32)
                accs = [jnp.zeros((L,), jnp.float32) for _ in range(16)]
                for w in range(J // 2):
                    colw = _splat_i32(w)
                    we = jnp.zeros((L,), jnp.float32)
                    wo = jnp.zeros((L,), jnp.float32)
                    for c in range(8):
                        v = plsc.load_gather(gbuf, [rows[c], colw])
                        pa, pb = plsc.unpack(
                            plsc.bitcast(v, jnp.bfloat16),
                            format=plsc.PackFormat.INTERLEAVED,
                            preferred_element_type=jnp.float32)
                        we = we + coefs[c] * pa
                        wo = wo + coefs[c] * pb
                    for j, wj in ((2 * w, we), (2 * w + 1, wo)):
                        s = s + wj
                        tfsr = tfs_v[pl.ds(j * 16, 16)]
                        for a in range(16):
                            accs[a] = accs[a] + wj * _vgather(tfsr, splats[a])
                inv = 1.0 / s
                wtf = [acc * inv for acc in accs]
                pos = cb + oo
                blk = pos // 128
                m0 = pos % 128
                for a in range(4):
                    for b in range(4):
                        wtf_v[pl.ds(a * PT * 4 + blk * 512 + b * 128 + m0,
                                    L)] = wtf[4 * a + b]
                for a in range(3):
                    pd = (wtf[4 * a] * px + wtf[4 * a + 1] * py
                          + wtf[4 * a + 2] * pz + wtf[4 * a + 3])
                    pdef_v[pl.ds(a * PT + pos, L)] = pd
                return c2

            lax.fori_loop(0, G, p2, 0)
            return carry

        lax.fori_loop(0, NCHUNK, chunk, 0)

        for c in range(3):
            pltpu.sync_copy(pdef_v.at[pl.ds(c * PT, PT)],
                            pdef_hbm.at[pl.ds(c * n + base, PT)])
        for a in range(4):
            pltpu.sync_copy(wtf_v.at[pl.ds(a * PT * 4, PT * 4)],
                            wtf_hbm.at[pl.ds(a * 4 * n + base * 4, PT * 4)])

    mesh = plsc.VectorSubcoreMesh(core_axis_name="c", subcore_axis_name="s",
                                  num_cores=NC, num_subcores=NS)
    kern = pl.kernel(
        body,
        out_type=(jax.ShapeDtypeStruct((n * 3,), f32),
                  jax.ShapeDtypeStruct((n * 16,), f32)),
        mesh=mesh,
        scratch_types=[
            pltpu.VMEM((so_flat.shape[0],), f32),
            pltpu.VMEM((J * 16,), f32),
            pltpu.VMEM((16,), f32),
            pltpu.VMEM((3 * PT,), f32),
            pltpu.VMEM((3 * PT,), f32),
            pltpu.VMEM((3 * PT,), jnp.int32),
            pltpu.VMEM((8 * C,), jnp.int32),
            pltpu.VMEM((8 * C, JP // 2), f32),
            pltpu.VMEM((6 * C,), f32),
            pltpu.VMEM((PT * 3,), f32),
            pltpu.VMEM((PT * 16,), f32),
            pltpu.SemaphoreType.DMA,
        ],
        compiler_params=pltpu.CompilerParams(needs_layout_passes=False,
                                             use_tc_tiling_on_sc=False),
    )
    return kern(vox_t, pts3, bar3, idx3, so_flat, tfs2, cst)


def _mask_body(q_ref, t_ref, tt_ref, o_ref):
    q = q_ref[...]
    t3 = t_ref[...]
    tt = tt_ref[...]
    qq = jnp.sum(q * q, axis=0, keepdims=True)
    qt = lax.dot_general(t3, q, (((0,), (0,)), ((), ())),
                         preferred_element_type=jnp.float32)
    d2 = qq - 2.0 * qt + tt
    m = jnp.min(d2, axis=0, keepdims=True)
    o_ref[...] = jnp.where(m < jnp.float32(THRESHOLD * THRESHOLD), 1.0, 0.0)


def _mask_call(q2, t3, ttc, n):
    BN = 2048
    TPAD = t3.shape[1]
    return pl.pallas_call(
        _mask_body,
        out_shape=jax.ShapeDtypeStruct((1, n), jnp.float32),
        grid=(n // BN,),
        in_specs=[
            pl.BlockSpec((3, BN), lambda i: (0, i)),
            pl.BlockSpec((3, TPAD), lambda i: (0, 0)),
            pl.BlockSpec((TPAD, 1), lambda i: (0, 0)),
        ],
        out_specs=pl.BlockSpec((1, BN), lambda i: (0, i)),
    )(q2, t3, ttc)


@jax.jit
def kernel(pts, vs_template, shape_offset, init_bar, tfs, lbs_voxel, scale,
           offset, init_idx):
    b, n, _ = pts.shape
    V = shape_offset.shape[1]

    vox = lbs_voxel[0].reshape(J, RES ** 3)
    au = lax.bitcast_convert_type(vox[0::2].astype(jnp.bfloat16),
                                  jnp.uint16).astype(jnp.uint32)
    bu = lax.bitcast_convert_type(vox[1::2].astype(jnp.bfloat16),
                                  jnp.uint16).astype(jnp.uint32)
    wrd = lax.bitcast_convert_type(au | (bu << 16), jnp.float32)
    vox_t = jnp.pad(jnp.transpose(wrd), ((0, 0), (0, JP // 2 - J // 2)))
    q2 = jnp.transpose(pts[0])
    pts3 = q2.reshape(-1)
    bar3 = jnp.transpose(init_bar[0]).reshape(-1)
    idx3 = jnp.transpose(init_idx.astype(jnp.int32)).reshape(-1)
    so3 = jnp.pad(jnp.transpose(shape_offset[0]).reshape(-1), (0, 2))
    tfs2 = tfs[0].reshape(-1)
    cst = jnp.concatenate([scale, offset, jnp.zeros((10,), jnp.float32)])

    tmpl = vs_template[0, ::10]
    T = tmpl.shape[0]
    TPAD = 768
    t3 = jnp.full((3, TPAD), 1e9, jnp.float32)
    t3 = t3.at[:, :T].set(tmpl.T)
    ttc = jnp.sum(t3 * t3, axis=0)[:, None]

    pdef_flat, wtf_flat = _sc_call(vox_t, pts3, bar3, idx3, so3, tfs2,
                                   cst, n, V)
    maskf = _mask_call(q2, t3, ttc, n)

    w_tf = (wtf_flat.reshape(4, n // 128, 4, 128)
            .transpose(1, 3, 0, 2).reshape(1, n, 4, 4))
    pts_cano_all = (pdef_flat.reshape(3, n).transpose(1, 0)
                    .reshape(1, n, 1, 3))
    mask = (maskf > 0.5).reshape(1, n, 1)
    return pts_cano_all, mask, w_tf


# speedup vs baseline: 4.0774x; 4.0774x over previous
import functools
import jax
import jax.numpy as jnp
from jax import lax
from jax.experimental import pallas as pl
from jax.experimental.pallas import tpu as pltpu
from jax.experimental.pallas import tpu_sc as plsc

THRESHOLD = 0.12
RES = 64
NC, NS, L = 2, 16, 16
NW = NC * NS
J = 24
JP = 32
C = 128
G = C // L

_CORNERS = [(dx, dy, dz) for dx in (0, 1) for dy in (0, 1) for dz in (0, 1)]


def _splat_i32(x):
    return jnp.full((L,), x, jnp.int32)


_GDN = lax.GatherDimensionNumbers(offset_dims=(), collapsed_slice_dims=(0,),
                                  start_index_map=(0,))


def _vgather(vec, idx):
    return lax.gather(vec, idx[:, None], _GDN, (1,),
                      mode=lax.GatherScatterMode.PROMISE_IN_BOUNDS)


def _gather_rows(table_hbm, idx_ref, dst_ref, sem):
    return pltpu.make_async_copy(table_hbm.at[idx_ref], dst_ref, sem)


def _worker_id():
    return lax.axis_index("s") * NC + lax.axis_index("c")


def _sc_call(vox_t, pts3, bar3, idx3, so_flat, tfs2, cst, n, V):
    PT = n // NW
    NCHUNK = PT // C
    PB = PT // 128
    f32 = jnp.float32

    def body(vox_hbm, pts_hbm, bar_hbm, idx_hbm, so_hbm, tfs_hbm, cst_hbm,
             pdef_hbm, wtf_hbm,
             so_v, tfs_v, cst_v, ptsv, barv, idxv, idxbuf, gbuf, fbuf,
             pdef_v, wtf_v, sem):
        wid = _worker_id()
        base = wid * PT

        pltpu.sync_copy(so_hbm, so_v)
        pltpu.sync_copy(tfs_hbm, tfs_v)
        pltpu.sync_copy(cst_hbm, cst_v)
        for c in range(3):
            pltpu.sync_copy(pts_hbm.at[pl.ds(c * n + base, PT)],
                            ptsv.at[pl.ds(c * PT, PT)])
            pltpu.sync_copy(bar_hbm.at[pl.ds(c * n + base, PT)],
                            barv.at[pl.ds(c * PT, PT)])
            pltpu.sync_copy(idx_hbm.at[pl.ds(c * n + base, PT)],
                            idxv.at[pl.ds(c * PT, PT)])

        iota = lax.iota(jnp.int32, L)
        splats = [_splat_i32(a) for a in range(16)]
        cstv = cst_v[pl.ds(0, L)]
        sx = _vgather(cstv, splats[0])
        sy = _vgather(cstv, splats[1])
        sz = _vgather(cstv, splats[2])
        ox = _vgather(cstv, splats[3])
        oy = _vgather(cstv, splats[4])
        oz = _vgather(cstv, splats[5])

        def axis_prep(p, s, o):
            g = jnp.clip(p * s + o, 0.0, 1.0) * jnp.float32(RES - 1)
            gi = jnp.minimum(g.astype(jnp.int32), RES - 2)
            return gi, g - gi.astype(jnp.float32)

        def chunk(t, carry):
            cb = t * C

            def p1(i, c2):
                o = cb + i * L
                qx = ptsv[pl.ds(o, L)]
                qy = ptsv[pl.ds(PT + o, L)]
                qz = ptsv[pl.ds(2 * PT + o, L)]
                b0 = barv[pl.ds(o, L)]
                b1 = barv[pl.ds(PT + o, L)]
                b2 = barv[pl.ds(2 * PT + o, L)]
                i0 = idxv[pl.ds(o, L)]
                i1 = idxv[pl.ds(PT + o, L)]
                i2 = idxv[pl.ds(2 * PT + o, L)]
                px = qx + (b0 * plsc.load_gather(so_v, [i0])
                           + b1 * plsc.load_gather(so_v, [i1])
                           + b2 * plsc.load_gather(so_v, [i2]))
                py = qy + (b0 * plsc.load_gather(so_v, [i0 + V])
                           + b1 * plsc.load_gather(so_v, [i1 + V])
                           + b2 * plsc.load_gather(so_v, [i2 + V]))
                pz = qz + (b0 * plsc.load_gather(so_v, [i0 + 2 * V])
                           + b1 * plsc.load_gather(so_v, [i1 + 2 * V])
                           + b2 * plsc.load_gather(so_v, [i2 + 2 * V]))
                gix, fx = axis_prep(px, sx, ox)
                giy, fy = axis_prep(py, sy, oy)
                giz, fz = axis_prep(pz, sz, oz)
                rowb = (gix * RES + giy) * RES + giz
                oo = i * L
                for c, (dx, dy, dz) in enumerate(_CORNERS):
                    idxbuf[pl.ds(c * C + oo, L)] = rowb + (dx * RES * RES
                                                           + dy * RES + dz)
                fbuf[pl.ds(oo, L)] = fx
                fbuf[pl.ds(C + oo, L)] = fy
                fbuf[pl.ds(2 * C + oo, L)] = fz
                fbuf[pl.ds(3 * C + oo, L)] = px
                fbuf[pl.ds(4 * C + oo, L)] = py
                fbuf[pl.ds(5 * C + oo, L)] = pz
                return c2

            lax.fori_loop(0, G, p1, 0)

            cps = []
            for c in range(8):
                cps.append(_gather_rows(
                    vox_hbm, idxbuf.at[pl.ds(c * C, C)],
                    gbuf.at[pl.ds(c * C, C)], sem))
            for cp in cps:
                cp.start()
            for cp in cps:
                cp.wait()

            def p2(i, c2):
                oo = i * L
                fx = fbuf[pl.ds(oo, L)]
                fy = fbuf[pl.ds(C + oo, L)]
                fz = fbuf[pl.ds(2 * C + oo, L)]
                px = fbuf[pl.ds(3 * C + oo, L)]
                py = fbuf[pl.ds(4 * C + oo, L)]
                pz = fbuf[pl.ds(5 * C + oo, L)]
                ex = 1.0 - fx
                ey = 1.0 - fy
                ez = 1.0 - fz
                cxy = [ex * ey, ex * fy, fx * ey, fx * fy]
                coefs = []
                for c, (dx, dy, dz) in enumerate(_CORNERS):
                    coefs.append(cxy[dx * 2 + dy] * (fz if dz else ez))
                rows = [iota + (oo + c * C) for c in range(8)]
                s = jnp.full((L,), 1e-8, jnp.float32)
                accs = [jnp.zeros((L,), jnp.float32) for _ in range(16)]
                for w in range(J // 2):
                    colw = _splat_i32(w)
                    we = jnp.zeros((L,), jnp.float32)
                    wo = jnp.zeros((L,), jnp.float32)
                    for c in range(8):
                        v = plsc.load_gather(gbuf, [rows[c], colw])
                        pa, pb = plsc.unpack(
                            plsc.bitcast(v, jnp.bfloat16),
                            format=plsc.PackFormat.INTERLEAVED,
                            preferred_element_type=jnp.float32)
                        we = we + coefs[c] * pa
                        wo = wo + coefs[c] * pb
                    for j, wj in ((2 * w, we), (2 * w + 1, wo)):
                        s = s + wj
                        tfsr = tfs_v[pl.ds(j * 16, 16)]
                        for a in range(16):
                            accs[a] = accs[a] + wj * _vgather(tfsr, splats[a])
                inv = 1.0 / s
                wtf = [acc * inv for acc in accs]
                pos = cb + oo
                blk = pos // 128
                m0 = pos % 128
                for a in range(4):
                    for b in range(4):
                        wtf_v[pl.ds(a * PT * 4 + blk * 512 + b * 128 + m0,
                                    L)] = wtf[4 * a + b]
                for a in range(3):
                    pd = (wtf[4 * a] * px + wtf[4 * a + 1] * py
                          + wtf[4 * a + 2] * pz + wtf[4 * a + 3])
                    pdef_v[pl.ds(a * PT + pos, L)] = pd
                return c2

            lax.fori_loop(0, G, p2, 0)
            return carry

        lax.fori_loop(0, NCHUNK, chunk, 0)

        for c in range(3):
            pltpu.sync_copy(pdef_v.at[pl.ds(c * PT, PT)],
                            pdef_hbm.at[pl.ds(c * n + base, PT)])
        for a in range(4):
            pltpu.sync_copy(wtf_v.at[pl.ds(a * PT * 4, PT * 4)],
                            wtf_hbm.at[pl.ds(a * 4 * n + base * 4, PT * 4)])

    mesh = plsc.VectorSubcoreMesh(core_axis_name="c", subcore_axis_name="s",
                                  num_cores=NC, num_subcores=NS)
    kern = pl.kernel(
        body,
        out_type=(jax.ShapeDtypeStruct((n * 3,), f32),
                  jax.ShapeDtypeStruct((n * 16,), f32)),
        mesh=mesh,
        scratch_types=[
            pltpu.VMEM((so_flat.shape[0],), f32),
            pltpu.VMEM((J * 16,), f32),
            pltpu.VMEM((16,), f32),
            pltpu.VMEM((3 * PT,), f32),
            pltpu.VMEM((3 * PT,), f32),
            pltpu.VMEM((3 * PT,), jnp.int32),
            pltpu.VMEM((8 * C,), jnp.int32),
            pltpu.VMEM((8 * C, JP // 2), f32),
            pltpu.VMEM((6 * C,), f32),
            pltpu.VMEM((PT * 3,), f32),
            pltpu.VMEM((PT * 16,), f32),
            pltpu.SemaphoreType.DMA,
        ],
        compiler_params=pltpu.CompilerParams(needs_layout_passes=False,
                                             use_tc_tiling_on_sc=False),
    )
    return kern(vox_t, pts3, bar3, idx3, so_flat, tfs2, cst)


def _mask_body(q_ref, t_ref, tt_ref, o_ref):
    q = q_ref[...]
    t3 = t_ref[...]
    tt = tt_ref[...]
    qq = jnp.sum(q * q, axis=0, keepdims=True)
    qt = lax.dot_general(t3, q, (((0,), (0,)), ((), ())),
                         preferred_element_type=jnp.float32)
    d2 = qq - 2.0 * qt + tt
    m = jnp.min(d2, axis=0, keepdims=True)
    o_ref[...] = jnp.where(m < jnp.float32(THRESHOLD * THRESHOLD), 1.0, 0.0)


def _mask_call(q2, t3, ttc, n):
    BN = 2048
    TPAD = t3.shape[1]
    return pl.pallas_call(
        _mask_body,
        out_shape=jax.ShapeDtypeStruct((1, n), jnp.float32),
        grid=(n // BN,),
        in_specs=[
            pl.BlockSpec((3, BN), lambda i: (0, i)),
            pl.BlockSpec((3, TPAD), lambda i: (0, 0)),
            pl.BlockSpec((TPAD, 1), lambda i: (0, 0)),
        ],
        out_specs=pl.BlockSpec((1, BN), lambda i: (0, i)),
    )(q2, t3, ttc)


@jax.jit
def kernel(pts, vs_template, shape_offset, init_bar, tfs, lbs_voxel, scale,
           offset, init_idx):
    b, n, _ = pts.shape
    V = shape_offset.shape[1]

    vox_t = (jnp.transpose(lbs_voxel[0].astype(jnp.bfloat16), (1, 2, 3, 0))
             .reshape(RES ** 3, J))
    vox_t = jnp.pad(vox_t, ((0, 0), (0, JP - J)))
    vox_t = lax.bitcast_convert_type(vox_t.reshape(RES ** 3, JP // 2, 2),
                                     jnp.float32)
    q2 = jnp.transpose(pts[0])
    pts3 = q2.reshape(-1)
    bar3 = jnp.transpose(init_bar[0]).reshape(-1)
    idx3 = jnp.transpose(init_idx.astype(jnp.int32)).reshape(-1)
    so3 = jnp.pad(jnp.transpose(shape_offset[0]).reshape(-1), (0, 2))
    tfs2 = tfs[0].reshape(-1)
    cst = jnp.concatenate([scale, offset, jnp.zeros((10,), jnp.float32)])

    tmpl = vs_template[0, ::10]
    T = tmpl.shape[0]
    TPAD = 768
    t3 = jnp.full((3, TPAD), 1e9, jnp.float32)
    t3 = t3.at[:, :T].set(tmpl.T)
    ttc = jnp.sum(t3 * t3, axis=0)[:, None]

    pdef_flat, wtf_flat = _sc_call(vox_t, pts3, bar3, idx3, so3, tfs2,
                                   cst, n, V)
    maskf = _mask_call(q2, t3, ttc, n)

    w_tf = (wtf_flat.reshape(4, n // 128, 4, 128)
            .transpose(1, 3, 0, 2).reshape(1, n, 4, 4))
    pts_cano_all = (pdef_flat.reshape(3, n).transpose(1, 0)
                    .reshape(1, n, 1, 3))
    mask = (maskf > 0.5).reshape(1, n, 1)
    return pts_cano_all, mask, w_tf


# speedup vs baseline: 4.2747x; 1.0484x over previous
import functools
import jax
import jax.numpy as jnp
from jax import lax
from jax.experimental import pallas as pl
from jax.experimental.pallas import tpu as pltpu
from jax.experimental.pallas import tpu_sc as plsc

THRESHOLD = 0.12
RES = 64
NC, NS, L = 2, 16, 16
NW = NC * NS
J = 24
JP = 32
C = 128
G = C // L

_CORNERS = [(dx, dy, dz) for dx in (0, 1) for dy in (0, 1) for dz in (0, 1)]


def _splat_i32(x):
    return jnp.full((L,), x, jnp.int32)


_GDN = lax.GatherDimensionNumbers(offset_dims=(), collapsed_slice_dims=(0,),
                                  start_index_map=(0,))


def _vgather(vec, idx):
    return lax.gather(vec, idx[:, None], _GDN, (1,),
                      mode=lax.GatherScatterMode.PROMISE_IN_BOUNDS)


def _gather_rows(table_hbm, idx_ref, dst_ref, sem):
    return pltpu.make_async_copy(table_hbm.at[idx_ref], dst_ref, sem)


def _worker_id():
    return lax.axis_index("s") * NC + lax.axis_index("c")


def _sc_call(vox_t, pts3, bar3, idx3, so_flat, tfs2, cst, n, V):
    PT = n // NW
    NCHUNK = PT // C
    PB = PT // 128
    f32 = jnp.float32

    def body(vox_hbm, pts_hbm, bar_hbm, idx_hbm, so_hbm, tfs_hbm, cst_hbm,
             pdef_hbm, wtf_hbm,
             so_v, tfs_v, cst_v, ptsv, barv, idxv, idxbuf, gbuf, fbuf,
             pdef_v, wtf_v, semA, semB):
        wid = _worker_id()
        base = wid * PT

        pltpu.sync_copy(so_hbm, so_v)
        pltpu.sync_copy(tfs_hbm, tfs_v)
        pltpu.sync_copy(cst_hbm, cst_v)
        for c in range(3):
            pltpu.sync_copy(pts_hbm.at[pl.ds(c * n + base, PT)],
                            ptsv.at[pl.ds(c * PT, PT)])
            pltpu.sync_copy(bar_hbm.at[pl.ds(c * n + base, PT)],
                            barv.at[pl.ds(c * PT, PT)])
            pltpu.sync_copy(idx_hbm.at[pl.ds(c * n + base, PT)],
                            idxv.at[pl.ds(c * PT, PT)])

        iota = lax.iota(jnp.int32, L)
        splats = [_splat_i32(a) for a in range(16)]
        cstv = cst_v[pl.ds(0, L)]
        sx = _vgather(cstv, splats[0])
        sy = _vgather(cstv, splats[1])
        sz = _vgather(cstv, splats[2])
        ox = _vgather(cstv, splats[3])
        oy = _vgather(cstv, splats[4])
        oz = _vgather(cstv, splats[5])

        def axis_prep(p, s, o):
            g = jnp.clip(p * s + o, 0.0, 1.0) * jnp.float32(RES - 1)
            gi = jnp.minimum(g.astype(jnp.int32), RES - 2)
            return gi, g - gi.astype(jnp.float32)

        def run_p1(t, kb):
            cb = t * C
            ib = kb * 8 * C
            fb = kb * 6 * C

            def p1(i, c2):
                o = cb + i * L
                qx = ptsv[pl.ds(o, L)]
                qy = ptsv[pl.ds(PT + o, L)]
                qz = ptsv[pl.ds(2 * PT + o, L)]
                b0 = barv[pl.ds(o, L)]
                b1 = barv[pl.ds(PT + o, L)]
                b2 = barv[pl.ds(2 * PT + o, L)]
                i0 = idxv[pl.ds(o, L)]
                i1 = idxv[pl.ds(PT + o, L)]
                i2 = idxv[pl.ds(2 * PT + o, L)]
                px = qx + (b0 * plsc.load_gather(so_v, [i0])
                           + b1 * plsc.load_gather(so_v, [i1])
                           + b2 * plsc.load_gather(so_v, [i2]))
                py = qy + (b0 * plsc.load_gather(so_v, [i0 + V])
                           + b1 * plsc.load_gather(so_v, [i1 + V])
                           + b2 * plsc.load_gather(so_v, [i2 + V]))
                pz = qz + (b0 * plsc.load_gather(so_v, [i0 + 2 * V])
                           + b1 * plsc.load_gather(so_v, [i1 + 2 * V])
                           + b2 * plsc.load_gather(so_v, [i2 + 2 * V]))
                gix, fx = axis_prep(px, sx, ox)
                giy, fy = axis_prep(py, sy, oy)
                giz, fz = axis_prep(pz, sz, oz)
                rowb = (gix * RES + giy) * RES + giz
                oo = i * L
                for c, (dx, dy, dz) in enumerate(_CORNERS):
                    idxbuf[pl.ds(ib + c * C + oo, L)] = rowb + (dx * RES * RES
                                                                + dy * RES + dz)
                fbuf[pl.ds(fb + oo, L)] = fx
                fbuf[pl.ds(fb + C + oo, L)] = fy
                fbuf[pl.ds(fb + 2 * C + oo, L)] = fz
                fbuf[pl.ds(fb + 3 * C + oo, L)] = px
                fbuf[pl.ds(fb + 4 * C + oo, L)] = py
                fbuf[pl.ds(fb + 5 * C + oo, L)] = pz
                return c2

            lax.fori_loop(0, G, p1, 0)

        def _cps(kb, sem):
            return [_gather_rows(
                vox_hbm, idxbuf.at[pl.ds(kb * 8 * C + c * C, C)],
                gbuf.at[pl.ds(kb * 8 * C + c * C, C)], sem)
                for c in range(8)]

        def fire(kb, sem):
            for cp in _cps(kb, sem):
                cp.start()

        def drain(kb, sem):
            for cp in _cps(kb, sem):
                cp.wait()

        def run_p2(t, kb):
            cb = t * C
            gb = kb * 8 * C
            fb = kb * 6 * C

            def p2(i, c2):
                oo = i * L
                fx = fbuf[pl.ds(fb + oo, L)]
                fy = fbuf[pl.ds(fb + C + oo, L)]
                fz = fbuf[pl.ds(fb + 2 * C + oo, L)]
                px = fbuf[pl.ds(fb + 3 * C + oo, L)]
                py = fbuf[pl.ds(fb + 4 * C + oo, L)]
                pz = fbuf[pl.ds(fb + 5 * C + oo, L)]
                ex = 1.0 - fx
                ey = 1.0 - fy
                ez = 1.0 - fz
                cxy = [ex * ey, ex * fy, fx * ey, fx * fy]
                coefs = []
                for c, (dx, dy, dz) in enumerate(_CORNERS):
                    coefs.append(cxy[dx * 2 + dy] * (fz if dz else ez))
                rows = [iota + (gb + oo + c * C) for c in range(8)]
                s = jnp.full((L,), 1e-8, jnp.float32)
                accs = [jnp.zeros((L,), jnp.float32) for _ in range(16)]
                for w in range(J // 2):
                    colw = _splat_i32(w)
                    we = jnp.zeros((L,), jnp.float32)
                    wo = jnp.zeros((L,), jnp.float32)
                    for c in range(8):
                        v = plsc.load_gather(gbuf, [rows[c], colw])
                        pa, pb = plsc.unpack(
                            plsc.bitcast(v, jnp.bfloat16),
                            format=plsc.PackFormat.INTERLEAVED,
                            preferred_element_type=jnp.float32)
                        we = we + coefs[c] * pa
                        wo = wo + coefs[c] * pb
                    for j, wj in ((2 * w, we), (2 * w + 1, wo)):
                        s = s + wj
                        tfsr = tfs_v[pl.ds(j * 16, 16)]
                        for a in range(16):
                            accs[a] = accs[a] + wj * _vgather(tfsr, splats[a])
                inv = 1.0 / s
                wtf = [acc * inv for acc in accs]
                pos = cb + oo
                blk = pos // 128
                m0 = pos % 128
                for a in range(4):
                    for b in range(4):
                        wtf_v[pl.ds(a * PT * 4 + blk * 512 + b * 128 + m0,
                                    L)] = wtf[4 * a + b]
                for a in range(3):
                    pd = (wtf[4 * a] * px + wtf[4 * a + 1] * py
                          + wtf[4 * a + 2] * pz + wtf[4 * a + 3])
                    pdef_v[pl.ds(a * PT + pos, L)] = pd
                return c2

            lax.fori_loop(0, G, p2, 0)

        run_p1(0, 0)
        fire(0, semA)
        if NCHUNK == 1:
            drain(0, semA)
            run_p2(0, 0)
        else:
            def pair(u, carry):
                t0 = 2 * u
                t1 = t0 + 1
                run_p1(t1, 1)
                fire(1, semB)
                drain(0, semA)
                run_p2(t0, 0)
                tn = jnp.minimum(t0 + 2, NCHUNK - 1)
                run_p1(tn, 0)
                fire(0, semA)
                drain(1, semB)
                run_p2(t1, 1)
                return carry

            lax.fori_loop(0, NCHUNK // 2, pair, 0)
            drain(0, semA)
            if NCHUNK % 2 == 1:
                run_p2(NCHUNK - 1, 0)

        for c in range(3):
            pltpu.sync_copy(pdef_v.at[pl.ds(c * PT, PT)],
                            pdef_hbm.at[pl.ds(c * n + base, PT)])
        for a in range(4):
            pltpu.sync_copy(wtf_v.at[pl.ds(a * PT * 4, PT * 4)],
                            wtf_hbm.at[pl.ds(a * 4 * n + base * 4, PT * 4)])

    mesh = plsc.VectorSubcoreMesh(core_axis_name="c", subcore_axis_name="s",
                                  num_cores=NC, num_subcores=NS)
    kern = pl.kernel(
        body,
        out_type=(jax.ShapeDtypeStruct((n * 3,), f32),
                  jax.ShapeDtypeStruct((n * 16,), f32)),
        mesh=mesh,
        scratch_types=[
            pltpu.VMEM((so_flat.shape[0],), f32),
            pltpu.VMEM((J * 16,), f32),
            pltpu.VMEM((16,), f32),
            pltpu.VMEM((3 * PT,), f32),
            pltpu.VMEM((3 * PT,), f32),
            pltpu.VMEM((3 * PT,), jnp.int32),
            pltpu.VMEM((2 * 8 * C,), jnp.int32),
            pltpu.VMEM((2 * 8 * C, JP // 2), f32),
            pltpu.VMEM((2 * 6 * C,), f32),
            pltpu.VMEM((PT * 3,), f32),
            pltpu.VMEM((PT * 16,), f32),
            pltpu.SemaphoreType.DMA,
            pltpu.SemaphoreType.DMA,
        ],
        compiler_params=pltpu.CompilerParams(needs_layout_passes=False,
                                             use_tc_tiling_on_sc=False),
    )
    return kern(vox_t, pts3, bar3, idx3, so_flat, tfs2, cst)


def _mask_body(q_ref, t_ref, tt_ref, o_ref):
    q = q_ref[...]
    t3 = t_ref[...]
    tt = tt_ref[...]
    qq = jnp.sum(q * q, axis=0, keepdims=True)
    qt = lax.dot_general(t3, q, (((0,), (0,)), ((), ())),
                         preferred_element_type=jnp.float32)
    d2 = qq - 2.0 * qt + tt
    m = jnp.min(d2, axis=0, keepdims=True)
    o_ref[...] = jnp.where(m < jnp.float32(THRESHOLD * THRESHOLD), 1.0, 0.0)


def _mask_call(q2, t3, ttc, n):
    BN = 2048
    TPAD = t3.shape[1]
    return pl.pallas_call(
        _mask_body,
        out_shape=jax.ShapeDtypeStruct((1, n), jnp.float32),
        grid=(n // BN,),
        in_specs=[
            pl.BlockSpec((3, BN), lambda i: (0, i)),
            pl.BlockSpec((3, TPAD), lambda i: (0, 0)),
            pl.BlockSpec((TPAD, 1), lambda i: (0, 0)),
        ],
        out_specs=pl.BlockSpec((1, BN), lambda i: (0, i)),
    )(q2, t3, ttc)


@jax.jit
def kernel(pts, vs_template, shape_offset, init_bar, tfs, lbs_voxel, scale,
           offset, init_idx):
    b, n, _ = pts.shape
    V = shape_offset.shape[1]

    vox_t = (jnp.transpose(lbs_voxel[0].astype(jnp.bfloat16), (1, 2, 3, 0))
             .reshape(RES ** 3, J))
    vox_t = jnp.pad(vox_t, ((0, 0), (0, JP - J)))
    vox_t = lax.bitcast_convert_type(vox_t.reshape(RES ** 3, JP // 2, 2),
                                     jnp.float32)
    q2 = jnp.transpose(pts[0])
    pts3 = q2.reshape(-1)
    bar3 = jnp.transpose(init_bar[0]).reshape(-1)
    idx3 = jnp.transpose(init_idx.astype(jnp.int32)).reshape(-1)
    so3 = jnp.pad(jnp.transpose(shape_offset[0]).reshape(-1), (0, 2))
    tfs2 = tfs[0].reshape(-1)
    cst = jnp.concatenate([scale, offset, jnp.zeros((10,), jnp.float32)])

    tmpl = vs_template[0, ::10]
    T = tmpl.shape[0]
    TPAD = 768
    t3 = jnp.full((3, TPAD), 1e9, jnp.float32)
    t3 = t3.at[:, :T].set(tmpl.T)
    ttc = jnp.sum(t3 * t3, axis=0)[:, None]

    pdef_flat, wtf_flat = _sc_call(vox_t, pts3, bar3, idx3, so3, tfs2,
                                   cst, n, V)
    maskf = _mask_call(q2, t3, ttc, n)

    w_tf = (wtf_flat.reshape(4, n // 128, 4, 128)
            .transpose(1, 3, 0, 2).reshape(1, n, 4, 4))
    pts_cano_all = (pdef_flat.reshape(3, n).transpose(1, 0)
                    .reshape(1, n, 1, 3))
    mask = (maskf > 0.5).reshape(1, n, 1)
    return pts_cano_all, mask, w_tf
